# baseline (speedup 1.0000x reference)
import functools

import jax
import jax.numpy as jnp
from jax import lax
from jax.experimental import pallas as pl
from jax.experimental.pallas import tpu as pltpu
from jax.experimental.pallas import tpu_sc as plsc

_LANES = 16
_RB = 128


@functools.cache
def _make_gather(R: int, S: int, D: int):
    scale = float(D) ** 0.5
    info = plsc.get_sparse_core_info()
    nw = info.num_cores * info.num_subcores
    n_jb = R // _RB
    n_items = S * n_jb
    items_per_w = n_items // nw
    assert n_items % nw == 0

    mesh = plsc.VectorSubcoreMesh(core_axis_name="c", subcore_axis_name="s")

    @functools.partial(
        pl.kernel,
        mesh=mesh,
        out_type=jax.ShapeDtypeStruct((S, D, R), jnp.float32),
        scratch_types=[
            pltpu.VMEM((_RB,), jnp.int32),
            pltpu.VMEM((_RB,), jnp.int32),
            pltpu.VMEM((_RB,), jnp.int32),
            pltpu.VMEM((_RB, 2 * D), jnp.float32),
            pltpu.VMEM((D, _RB), jnp.float32),
            pltpu.SemaphoreType.DMA,
        ],
        compiler_params=pltpu.CompilerParams(
            use_tc_tiling_on_sc=True, needs_layout_passes=False
        ),
    )
    def gather_kernel(
        xtf_hbm, pairs_hbm, out_hbm, idx_v, u_v, h_v, rows_v, trans_v, sem
    ):
        wid = lax.axis_index("s") * info.num_cores + lax.axis_index("c")
        it0 = wid * items_per_w
        iota = lax.iota(jnp.int32, _LANES)

        def item_body(k, carry):
            it = it0 + k
            si = it // n_jb
            jb = it - si * n_jb
            r0 = jb * _RB
            pltpu.sync_copy(xtf_hbm.at[pl.ds(si * R + r0, _RB)], idx_v)

            def prep_body(g, c2):
                sl = pl.ds(g * _LANES, _LANES)
                v = idx_v[sl]
                u_v[sl] = lax.shift_right_logical(v, 1)
                h_v[sl] = (v & 1) * D
                return c2

            lax.fori_loop(0, _RB // _LANES, prep_body, 0)
            pltpu.async_copy(pairs_hbm.at[u_v], rows_v, sem).wait()

            def trans_body(d, c2):
                for g in range(_RB // _LANES):
                    sl = pl.ds(g * _LANES, _LANES)
                    rowsel = iota + (g * _LANES)
                    colsel = h_v[sl] + d
                    vals = plsc.load_gather(rows_v, [rowsel, colsel])
                    trans_v[d, sl] = vals * scale
                return c2

            lax.fori_loop(0, D, trans_body, 0)
            pltpu.sync_copy(trans_v, out_hbm.at[si, :, pl.ds(r0, _RB)])
            return carry

        lax.fori_loop(0, items_per_w, item_body, 0)

    return gather_kernel


def kernel(x, table):
    R, S = x.shape
    V, D = table.shape
    xtf = jnp.transpose(x).reshape(S * R)
    pairs = table.reshape(V // 2, 2 * D)
    out_t = _make_gather(R, S, D)(xtf, pairs)
    return jnp.transpose(out_t, (2, 0, 1))


# speedup vs baseline: 2.2020x; 2.2020x over previous
import functools

import jax
import jax.numpy as jnp
from jax import lax
from jax.experimental import pallas as pl
from jax.experimental.pallas import tpu as pltpu
from jax.experimental.pallas import tpu_sc as plsc

_LANES = 16
_RB = 128


@functools.cache
def _make_gather(R: int, S: int, D: int):
    scale = float(D) ** 0.5
    info = plsc.get_sparse_core_info()
    nw = info.num_cores * info.num_subcores
    n_jb = R // _RB
    n_items = S * n_jb
    items_per_w = n_items // nw
    assert n_items % nw == 0 and items_per_w % 2 == 0

    mesh = plsc.VectorSubcoreMesh(core_axis_name="c", subcore_axis_name="s")

    @functools.partial(
        pl.kernel,
        mesh=mesh,
        out_type=jax.ShapeDtypeStruct((S, D, R), jnp.float32),
        scratch_types=[
            pltpu.VMEM((2, _RB), jnp.int32),
            pltpu.VMEM((2, _RB), jnp.int32),
            pltpu.VMEM((2, _RB), jnp.int32),
            pltpu.VMEM((2, _RB, 2 * D), jnp.float32),
            pltpu.VMEM((D, _RB), jnp.float32),
            pltpu.SemaphoreType.DMA,
            pltpu.SemaphoreType.DMA,
        ],
        compiler_params=pltpu.CompilerParams(
            use_tc_tiling_on_sc=True, needs_layout_passes=False
        ),
    )
    def gather_kernel(
        xtf_hbm, pairs_hbm, out_hbm, idx_v, u_v, h_v, rows_v, trans_v, sem0, sem1
    ):
        wid = lax.axis_index("s") * info.num_cores + lax.axis_index("c")
        it0 = wid * items_per_w
        iota = lax.iota(jnp.int32, _LANES)
        sems = (sem0, sem1)
        rowsel = [iota + (g * _LANES) for g in range(_RB // _LANES)]
        diag = [(iota + k) & (_LANES - 1) for k in range(_LANES)]

        def fetch_prep_fire(item, b):
            si = item // n_jb
            r0 = (item - si * n_jb) * _RB
            pltpu.sync_copy(xtf_hbm.at[pl.ds(si * R + r0, _RB)], idx_v.at[b])

            def prep_body(g, c2):
                sl = pl.ds(g * _LANES, _LANES)
                v = idx_v[b, sl]
                u_v[b, sl] = lax.shift_right_logical(v, 1)
                h_v[b, sl] = (v & 1) * D
                return c2

            lax.fori_loop(0, _RB // _LANES, prep_body, 0)
            pltpu.async_copy(pairs_hbm.at[u_v.at[b]], rows_v.at[b], sems[b])

        def process(item, b):
            pltpu.make_async_copy(
                pairs_hbm.at[pl.ds(0, _RB)], rows_v.at[b], sems[b]
            ).wait()
            rows_b = rows_v.at[b]

            def db_body(db, c2):
                d0 = db * _LANES
                for g in range(_RB // _LANES):
                    hg = h_v[b, pl.ds(g * _LANES, _LANES)]
                    colbase = hg + d0
                    for k in range(_LANES):
                        colsel = colbase + diag[k]
                        dvec = colsel - hg
                        vals = plsc.load_gather(rows_b, [rowsel[g], colsel])
                        plsc.store_scatter(
                            trans_v, [dvec, rowsel[g]], vals * scale
                        )
                return c2

            lax.fori_loop(0, D // _LANES, db_body, 0)
            si = item // n_jb
            r0 = (item - si * n_jb) * _RB
            pltpu.sync_copy(trans_v, out_hbm.at[si, :, pl.ds(r0, _RB)])

        fetch_prep_fire(it0, 0)

        def pair_body(k2, carry):
            base = it0 + 2 * k2
            fetch_prep_fire(jnp.minimum(base + 1, it0 + items_per_w - 1), 1)
            process(base, 0)
            fetch_prep_fire(jnp.minimum(base + 2, it0 + items_per_w - 1), 0)
            process(base + 1, 1)
            return carry

        lax.fori_loop(0, items_per_w // 2, pair_body, 0)
        pltpu.make_async_copy(
            pairs_hbm.at[pl.ds(0, _RB)], rows_v.at[0], sems[0]
        ).wait()

    return gather_kernel


def kernel(x, table):
    R, S = x.shape
    V, D = table.shape
    xtf = jnp.transpose(x).reshape(S * R)
    pairs = table.reshape(V // 2, 2 * D)
    out_t = _make_gather(R, S, D)(xtf, pairs)
    return jnp.transpose(out_t, (2, 0, 1))


# speedup vs baseline: 2.7690x; 1.2575x over previous
import functools

import jax
import jax.numpy as jnp
from jax import lax
from jax.experimental import pallas as pl
from jax.experimental.pallas import tpu as pltpu
from jax.experimental.pallas import tpu_sc as plsc

_LANES = 16
_RB = 128


@functools.cache
def _make_gather(R: int, S: int, D: int):
    scale = float(D) ** 0.5
    info = plsc.get_sparse_core_info()
    nw = info.num_cores * info.num_subcores
    n_jb = R // _RB
    n_items = S * n_jb
    items_per_w = n_items // nw
    assert n_items % nw == 0 and items_per_w % 2 == 0

    mesh = plsc.VectorSubcoreMesh(core_axis_name="c", subcore_axis_name="s")

    @functools.partial(
        pl.kernel,
        mesh=mesh,
        out_type=jax.ShapeDtypeStruct((S, D, R), jnp.float32),
        scratch_types=[
            pltpu.VMEM((2, _RB), jnp.int32),
            pltpu.VMEM((2, _RB), jnp.int32),
            pltpu.VMEM((2, _RB), jnp.int32),
            pltpu.VMEM((2, _RB, 2 * D), jnp.float32),
            pltpu.VMEM((D, _RB), jnp.float32),
            pltpu.SemaphoreType.DMA,
            pltpu.SemaphoreType.DMA,
        ],
        compiler_params=pltpu.CompilerParams(
            use_tc_tiling_on_sc=True,
            needs_layout_passes=False,
            disable_bounds_checks=True,
        ),
    )
    def gather_kernel(
        xtf_hbm, pairs_hbm, out_hbm, idx_v, u_v, h_v, rows_v, trans_v, sem0, sem1
    ):
        wid = lax.axis_index("s") * info.num_cores + lax.axis_index("c")
        it0 = wid * items_per_w
        iota = lax.iota(jnp.int32, _LANES)
        sems = (sem0, sem1)
        rowsel = [iota + (g * _LANES) for g in range(_RB // _LANES)]
        diag = [(iota + k) & (_LANES - 1) for k in range(_LANES)]

        def fetch_prep_fire(item, b):
            si = item // n_jb
            r0 = (item - si * n_jb) * _RB
            pltpu.sync_copy(xtf_hbm.at[pl.ds(si * R + r0, _RB)], idx_v.at[b])

            def prep_body(g, c2):
                sl = pl.ds(g * _LANES, _LANES)
                v = idx_v[b, sl]
                u_v[b, sl] = lax.shift_right_logical(v, 1)
                h_v[b, sl] = (v & 1) * D
                return c2

            lax.fori_loop(0, _RB // _LANES, prep_body, 0)
            pltpu.async_copy(pairs_hbm.at[u_v.at[b]], rows_v.at[b], sems[b])

        def process(item, b):
            pltpu.make_async_copy(
                pairs_hbm.at[pl.ds(0, _RB)], rows_v.at[b], sems[b]
            ).wait()
            rows_b = rows_v.at[b]

            def db_body(db, c2):
                d0 = db * _LANES
                for g in range(_RB // _LANES):
                    hg = h_v[b, pl.ds(g * _LANES, _LANES)]
                    colbase = hg + d0
                    colsels = [colbase + diag[k] for k in range(_LANES)]
                    vals = [
                        plsc.load_gather(rows_b, [rowsel[g], colsels[k]]) * scale
                        for k in range(_LANES)
                    ]
                    for k in range(_LANES):
                        plsc.store_scatter(
                            trans_v, [colsels[k] - hg, rowsel[g]], vals[k]
                        )
                return c2

            lax.fori_loop(0, D // _LANES, db_body, 0)
            si = item // n_jb
            r0 = (item - si * n_jb) * _RB
            pltpu.sync_copy(trans_v, out_hbm.at[si, :, pl.ds(r0, _RB)])

        fetch_prep_fire(it0, 0)

        def pair_body(k2, carry):
            base = it0 + 2 * k2
            fetch_prep_fire(jnp.minimum(base + 1, it0 + items_per_w - 1), 1)
            process(base, 0)
            fetch_prep_fire(jnp.minimum(base + 2, it0 + items_per_w - 1), 0)
            process(base + 1, 1)
            return carry

        lax.fori_loop(0, items_per_w // 2, pair_body, 0)
        pltpu.make_async_copy(
            pairs_hbm.at[pl.ds(0, _RB)], rows_v.at[0], sems[0]
        ).wait()

    return gather_kernel


def kernel(x, table):
    R, S = x.shape
    V, D = table.shape
    xtf = jnp.transpose(x).reshape(S * R)
    pairs = table.reshape(V // 2, 2 * D)
    out_t = _make_gather(R, S, D)(xtf, pairs)
    return jnp.transpose(out_t, (2, 0, 1))


# speedup vs baseline: 3.3433x; 1.2074x over previous
import functools

import jax
import jax.numpy as jnp
from jax import lax
from jax.experimental import pallas as pl
from jax.experimental.pallas import tpu as pltpu
from jax.experimental.pallas import tpu_sc as plsc

_LANES = 16
_RB = 128


@functools.cache
def _make_gather(R: int, S: int, D: int):
    scale = float(D) ** 0.5
    info = plsc.get_sparse_core_info()
    nw = info.num_cores * info.num_subcores
    n_jb = R // _RB
    n_items = S * n_jb
    items_per_w = n_items // nw
    assert n_items % nw == 0 and items_per_w % 2 == 0

    mesh = plsc.VectorSubcoreMesh(core_axis_name="c", subcore_axis_name="s")

    @functools.partial(
        pl.kernel,
        mesh=mesh,
        out_type=jax.ShapeDtypeStruct((S, D, R), jnp.float32),
        scratch_types=[
            pltpu.VMEM((2, _RB), jnp.int32),
            pltpu.VMEM((2, _RB), jnp.int32),
            pltpu.VMEM((2, _RB), jnp.int32),
            pltpu.VMEM((2, _RB, 2 * D), jnp.float32),
            pltpu.VMEM((2, D, _RB), jnp.float32),
            pltpu.SemaphoreType.DMA,
            pltpu.SemaphoreType.DMA,
            pltpu.SemaphoreType.DMA,
            pltpu.SemaphoreType.DMA,
        ],
        compiler_params=pltpu.CompilerParams(
            use_tc_tiling_on_sc=True,
            needs_layout_passes=False,
            disable_bounds_checks=True,
        ),
    )
    def gather_kernel(
        xtf_hbm,
        pairs_hbm,
        out_hbm,
        idx_v,
        u_v,
        h_v,
        rows_v,
        trans_v,
        sem0,
        sem1,
        sem_o0,
        sem_o1,
    ):
        wid = lax.axis_index("s") * info.num_cores + lax.axis_index("c")
        it0 = wid * items_per_w
        iota = lax.iota(jnp.int32, _LANES)
        sems = (sem0, sem1)
        osems = (sem_o0, sem_o1)
        rowsel = [iota + (g * _LANES) for g in range(_RB // _LANES)]
        diag = [(iota + k) & (_LANES - 1) for k in range(_LANES)]

        def fetch_prep_fire(item, b):
            si = item // n_jb
            r0 = (item - si * n_jb) * _RB
            pltpu.sync_copy(xtf_hbm.at[pl.ds(si * R + r0, _RB)], idx_v.at[b])

            def prep_body(g, c2):
                sl = pl.ds(g * _LANES, _LANES)
                v = idx_v[b, sl]
                u_v[b, sl] = lax.shift_right_logical(v, 1)
                h_v[b, sl] = (v & 1) * D
                return c2

            lax.fori_loop(0, _RB // _LANES, prep_body, 0)
            pltpu.async_copy(pairs_hbm.at[u_v.at[b]], rows_v.at[b], sems[b])

        def process(item, b, first):
            pltpu.make_async_copy(
                pairs_hbm.at[pl.ds(0, _RB)], rows_v.at[b], sems[b]
            ).wait()
            if not first:
                pltpu.make_async_copy(
                    out_hbm.at[0, :, pl.ds(0, _RB)], trans_v.at[b], osems[b]
                ).wait()
            rows_b = rows_v.at[b]
            trans_b = trans_v.at[b]

            def db_body(db, c2):
                d0 = db * _LANES
                for g in range(_RB // _LANES):
                    hg = h_v[b, pl.ds(g * _LANES, _LANES)]
                    colbase = hg + d0
                    colsels = [colbase + diag[k] for k in range(_LANES)]
                    vals = [
                        plsc.load_gather(rows_b, [rowsel[g], colsels[k]]) * scale
                        for k in range(_LANES)
                    ]
                    for k in range(_LANES):
                        plsc.store_scatter(
                            trans_b, [colsels[k] - hg, rowsel[g]], vals[k]
                        )
                return c2

            lax.fori_loop(0, D // _LANES, db_body, 0)
            si = item // n_jb
            r0 = (item - si * n_jb) * _RB
            pltpu.async_copy(trans_b, out_hbm.at[si, :, pl.ds(r0, _RB)], osems[b])

        last = it0 + items_per_w - 1
        fetch_prep_fire(it0, 0)
        fetch_prep_fire(it0 + 1, 1)
        process(it0, 0, True)
        fetch_prep_fire(it0 + 2, 0)
        process(it0 + 1, 1, True)

        def pair_body(k2, carry):
            base = it0 + 2 * k2
            fetch_prep_fire(jnp.minimum(base + 1, last), 1)
            process(base, 0, False)
            fetch_prep_fire(jnp.minimum(base + 2, last), 0)
            process(base + 1, 1, False)
            return carry

        lax.fori_loop(1, items_per_w // 2, pair_body, 0)
        pltpu.make_async_copy(
            pairs_hbm.at[pl.ds(0, _RB)], rows_v.at[0], sems[0]
        ).wait()
        for b in range(2):
            pltpu.make_async_copy(
                out_hbm.at[0, :, pl.ds(0, _RB)], trans_v.at[b], osems[b]
            ).wait()

    return gather_kernel


def kernel(x, table):
    R, S = x.shape
    V, D = table.shape
    xtf = jnp.transpose(x).reshape(S * R)
    pairs = table.reshape(V // 2, 2 * D)
    out_t = _make_gather(R, S, D)(xtf, pairs)
    return jnp.transpose(out_t, (2, 0, 1))
